# baseline (speedup 1.0000x reference)
import jax
import jax.numpy as jnp
from jax import lax
from jax.experimental import pallas as pl
from jax.experimental.pallas import tpu as pltpu
from jax.experimental.pallas import tpu_sc as plsc

N_USERS = 100000
N_ITEMS = 900000
EMB = 32
REGS = 1e-5
B = 16384

NC = 2
NS = 16
NW = NC * NS
PB = B // NW
CHUNK = 128
NCH = PB // CHUNK
NBLK = 250000


def _sc_body(tbl_hbm, u_idx_hbm, p_idx_hbm, n_idx_hbm,
             sc_hbm, sq_hbm,
             u_idx_v, p_idx_v, n_idx_v,
             uo_v, po_v, no_v,
             u_blk, p_blk, n_blk,
             sc_v, sq_v, sem):
    wid = lax.axis_index("s") * NC + lax.axis_index("c")

    pltpu.sync_copy(u_idx_hbm.at[wid], u_idx_v)
    pltpu.sync_copy(p_idx_hbm.at[wid], p_idx_v)
    pltpu.sync_copy(n_idx_hbm.at[wid], n_idx_v)
    for src, off in ((u_idx_v, uo_v), (p_idx_v, po_v), (n_idx_v, no_v)):
        for j in range(NCH):
            for v in range(CHUNK // 16):
                x = src[j, pl.ds(v * 16, 16)]
                off[j, pl.ds(v * 16, 16)] = (x & 3) * 32
                src[j, pl.ds(v * 16, 16)] = lax.shift_right_logical(x, 2)

    sq = jnp.zeros((16,), jnp.float32)
    for p in range(NCH):
        copies = []
        for idx_v, blk in ((u_idx_v, u_blk), (p_idx_v, p_blk), (n_idx_v, n_blk)):
            copies.append(pltpu.async_copy(
                tbl_hbm.at[idx_v.at[p]], blk, sem))
        for c in copies:
            c.wait()

        def group(g, sq):
            lanes = pl.ds(g * 16, 16)
            ku = lax.iota(jnp.int32, 16) + g * 16
            offu = uo_v[p, lanes]
            offp = po_v[p, lanes]
            offn = no_v[p, lanes]

            def dim(d, carry):
                acc, sq = carry
                vu = plsc.load_gather(u_blk, [ku, offu + d])
                vp = plsc.load_gather(p_blk, [ku, offp + d])
                vn = plsc.load_gather(n_blk, [ku, offn + d])
                return (acc + vu * (vp - vn),
                        sq + vu * vu + vp * vp + vn * vn)

            acc, sq = lax.fori_loop(
                0, EMB, dim, (jnp.zeros((16,), jnp.float32), sq))
            sc_v[p, lanes] = acc
            return sq

        sq = lax.fori_loop(0, CHUNK // 16, group, sq)

    sq_v[...] = sq
    pltpu.sync_copy(sc_v, sc_hbm.at[wid])
    pltpu.sync_copy(sq_v, sq_hbm.at[wid])


def _sc_call(tbl, u_idx, p_idx, n_idx):
    mesh = plsc.VectorSubcoreMesh(core_axis_name="c", subcore_axis_name="s")
    return pl.kernel(
        _sc_body,
        out_type=(
            jax.ShapeDtypeStruct((NW, NCH, CHUNK), jnp.float32),
            jax.ShapeDtypeStruct((NW, 16), jnp.float32),
        ),
        mesh=mesh,
        compiler_params=pltpu.CompilerParams(
            use_tc_tiling_on_sc=True, needs_layout_passes=False),
        scratch_types=[
            pltpu.VMEM((NCH, CHUNK), jnp.int32),
            pltpu.VMEM((NCH, CHUNK), jnp.int32),
            pltpu.VMEM((NCH, CHUNK), jnp.int32),
            pltpu.VMEM((NCH, CHUNK), jnp.int32),
            pltpu.VMEM((NCH, CHUNK), jnp.int32),
            pltpu.VMEM((NCH, CHUNK), jnp.int32),
            pltpu.VMEM((CHUNK, 128), jnp.float32),
            pltpu.VMEM((CHUNK, 128), jnp.float32),
            pltpu.VMEM((CHUNK, 128), jnp.float32),
            pltpu.VMEM((NCH, CHUNK), jnp.float32),
            pltpu.VMEM((16,), jnp.float32),
            pltpu.SemaphoreType.DMA,
        ],
    )(tbl, u_idx, p_idx, n_idx)


def _tc_epilogue_body(sc_ref, sq_ref, bpr_ref, reg_ref):
    d = sc_ref[...]
    logsig = -jnp.log1p(jnp.exp(-d))
    bpr_ref[...] = jnp.full((1, 1), -jnp.mean(logsig), jnp.float32)
    reg_ref[...] = jnp.full((1, 1), REGS * 0.5 * jnp.sum(sq_ref[...]),
                            jnp.float32)


def _tc_epilogue(sc, sq):
    return pl.pallas_call(
        _tc_epilogue_body,
        out_shape=(
            jax.ShapeDtypeStruct((1, 1), jnp.float32),
            jax.ShapeDtypeStruct((1, 1), jnp.float32),
        ),
    )(sc, sq)


@jax.jit
def kernel(user, pos_item, neg_item, table):
    tbl = table[:4 * NBLK].reshape(NBLK, 128)
    u_idx = user.reshape(NW, NCH, CHUNK)
    p_idx = pos_item.reshape(NW, NCH, CHUNK)
    n_idx = neg_item.reshape(NW, NCH, CHUNK)
    sc, sq = _sc_call(tbl, u_idx, p_idx, n_idx)
    bpr, reg = _tc_epilogue(sc.reshape(B // 128, 128), sq)
    return (bpr.reshape(()), reg.reshape(()))
